# baseline (speedup 1.0000x reference)
import jax
import jax.numpy as jnp
from jax.experimental import pallas as pl
from jax.experimental.pallas import tpu as pltpu

_B, _T, _N, _F, _L = 32, 12, 207, 64, 2
_P = 208
_BB = 8
_M = _BB * _P


def _attend(h, lhs, d_row):
    rhs = jnp.concatenate(
        [jnp.ones((1, _N), dtype=jnp.float32), d_row], axis=0)
    e = jnp.dot(lhs, rhs, preferred_element_type=jnp.float32)
    e = jnp.maximum(e, 0.2 * e)
    p = jnp.exp(e)
    rs = 1.0 / jnp.sum(p, axis=-1, keepdims=True)
    out = jnp.dot(p, h, preferred_element_type=jnp.float32) * rs
    return out, p, rs


def _garnn_kernel(x_ref, wi_ref, wh_ref,
                  out_ref, hid_ref, attn_i_ref, attn_h_ref,
                  h_scr, x2_scr, gs_scr, gin_scr, ghn_scr):
    t = pl.program_id(1)

    @pl.when(t == 0)
    def _():
        h_scr[...] = jnp.zeros_like(h_scr)
        x2_scr[...] = jnp.zeros_like(x2_scr)
        gs_scr[...] = jnp.zeros_like(gs_scr)
        gin_scr[...] = jnp.zeros_like(gin_scr)
        ghn_scr[...] = jnp.zeros_like(ghn_scr)

    last = t == _T - 1
    for i in range(_BB):
        x2_scr[i * _P:i * _P + _N] = x_ref[i, 0]
    x2 = x2_scr[...]
    for l in range(_L):
        hfull = h_scr[l]
        gi_f = jnp.dot(x2, wi_ref[l], preferred_element_type=jnp.float32)
        gh_f = jnp.dot(hfull, wh_ref[l], preferred_element_type=jnp.float32)
        e2i = gi_f[:, 3 * _F:3 * _F + 2]
        e2h = gh_f[:, 3 * _F:3 * _F + 2]
        ones_m = jnp.ones((_M, 1), dtype=jnp.float32)
        lhs_i = jnp.concatenate([e2i[:, 0:1], ones_m], axis=1)
        lhs_h = jnp.concatenate([e2h[:, 0:1], ones_m], axis=1)
        d_i = e2i[:, 1:2].reshape(_BB, _P)
        d_h = e2h[:, 1:2].reshape(_BB, _P)
        for i in range(_BB):
            lo = i * _P
            gi, p_i, rs_i = _attend(gi_f[lo:lo + _N],
                                    lhs_i[lo:lo + _N], d_i[i:i + 1, :_N])
            gh, p_h, rs_h = _attend(gh_f[lo:lo + _N],
                                    lhs_h[lo:lo + _N], d_h[i:i + 1, :_N])
            gs_scr[lo:lo + _N] = gi[:, :2 * _F] + gh[:, :2 * _F]
            gin_scr[lo:lo + _N] = gi[:, 2 * _F:3 * _F]
            ghn_scr[lo:lo + _N] = gh[:, 2 * _F:3 * _F]

            @pl.when(last)
            def _(i=i, l=l, p_i=p_i, rs_i=rs_i, p_h=p_h, rs_h=rs_h):
                attn_i_ref[i, l] = p_i * rs_i
                attn_h_ref[i, l] = p_h * rs_h
        s = jax.nn.sigmoid(gs_scr[...])
        r = s[:, :_F]
        z = s[:, _F:]
        n = jnp.tanh(gin_scr[...] + r * ghn_scr[...])
        h_new = (1.0 - z) * n + z * hfull
        h_scr[l] = h_new
        for i in range(_BB):
            lo = i * _P
            if l == _L - 1:
                out_ref[i, 0] = h_new[lo:lo + _N]

            @pl.when(last)
            def _(i=i, l=l, lo=lo, h_new=h_new):
                hid_ref[i, l] = h_new[lo:lo + _N]
        x2 = h_new


def kernel(input, Wi, Wh, ai, ah):
    Bb, Tt, Nn, Ff = input.shape
    Ll = Wi.shape[0]
    nb = Bb // _BB

    wai = jnp.einsum('lfg,lag->lfa', Wi, ai)
    wah = jnp.einsum('lfg,lag->lfa', Wh, ah)
    wie = jnp.concatenate([Wi, wai], axis=2)
    whe = jnp.concatenate([Wh, wah], axis=2)

    grid = (nb, Tt)
    out_shapes = (
        jax.ShapeDtypeStruct((Bb, Tt, Nn, Ff), jnp.float32),
        jax.ShapeDtypeStruct((Bb, Ll, Nn, Ff), jnp.float32),
        jax.ShapeDtypeStruct((Bb, Ll, Nn, Nn), jnp.float32),
        jax.ShapeDtypeStruct((Bb, Ll, Nn, Nn), jnp.float32),
    )
    in_specs = [
        pl.BlockSpec((_BB, 1, Nn, Ff), lambda b, t: (b, t, 0, 0)),
        pl.BlockSpec((Ll, Ff, 3 * Ff + 2), lambda b, t: (0, 0, 0)),
        pl.BlockSpec((Ll, Ff, 3 * Ff + 2), lambda b, t: (0, 0, 0)),
    ]
    out_specs = (
        pl.BlockSpec((_BB, 1, Nn, Ff), lambda b, t: (b, t, 0, 0)),
        pl.BlockSpec((_BB, Ll, Nn, Ff), lambda b, t: (b, 0, 0, 0)),
        pl.BlockSpec((_BB, Ll, Nn, Nn), lambda b, t: (b, 0, 0, 0)),
        pl.BlockSpec((_BB, Ll, Nn, Nn), lambda b, t: (b, 0, 0, 0)),
    )
    output, hidden, attn_i, attn_h = pl.pallas_call(
        _garnn_kernel,
        grid=grid,
        in_specs=in_specs,
        out_specs=out_specs,
        out_shape=out_shapes,
        scratch_shapes=[pltpu.VMEM((Ll, _M, Ff), jnp.float32),
                        pltpu.VMEM((_M, Ff), jnp.float32),
                        pltpu.VMEM((_M, 2 * Ff), jnp.float32),
                        pltpu.VMEM((_M, Ff), jnp.float32),
                        pltpu.VMEM((_M, Ff), jnp.float32)],
        compiler_params=pltpu.CompilerParams(
            dimension_semantics=("parallel", "arbitrary"),
        ),
    )(input, wie, whe)
    return (output, hidden, attn_i, attn_h)


# speedup vs baseline: 1.0031x; 1.0031x over previous
import jax
import jax.numpy as jnp
from jax.experimental import pallas as pl
from jax.experimental.pallas import tpu as pltpu

_B, _T, _N, _F, _L = 32, 12, 207, 64, 2
_P = 208
_BB = 4
_M = _BB * _P


def _attend(h, lhs, d_row):
    rhs = jnp.concatenate(
        [jnp.ones((1, _N), dtype=jnp.float32), d_row], axis=0)
    e = jnp.dot(lhs, rhs, preferred_element_type=jnp.float32)
    e = jnp.maximum(e, 0.2 * e)
    p = jnp.exp(e)
    rs = 1.0 / jnp.sum(p, axis=-1, keepdims=True)
    out = jnp.dot(p, h, preferred_element_type=jnp.float32) * rs
    return out, p, rs


def _garnn_kernel(x_ref, wi_ref, wh_ref,
                  out_ref, hid_ref, attn_i_ref, attn_h_ref,
                  h_scr, x2_scr, gs_scr, gin_scr, ghn_scr):
    h_scr[...] = jnp.zeros_like(h_scr)
    x2_scr[...] = jnp.zeros_like(x2_scr)

    def _step(t, carry):
        last = t == _T - 1
        for i in range(_BB):
            x2_scr[i * _P:i * _P + _N] = x_ref[i, t]
        x2 = x2_scr[...]
        for l in range(_L):
            hfull = h_scr[l]
            gi_f = jnp.dot(x2, wi_ref[l], preferred_element_type=jnp.float32)
            gh_f = jnp.dot(hfull, wh_ref[l], preferred_element_type=jnp.float32)
            e2i = gi_f[:, 3 * _F:3 * _F + 2]
            e2h = gh_f[:, 3 * _F:3 * _F + 2]
            ones_m = jnp.ones((_M, 1), dtype=jnp.float32)
            lhs_i = jnp.concatenate([e2i[:, 0:1], ones_m], axis=1)
            lhs_h = jnp.concatenate([e2h[:, 0:1], ones_m], axis=1)
            d_i = e2i[:, 1:2].reshape(_BB, _P)
            d_h = e2h[:, 1:2].reshape(_BB, _P)
            for i in range(_BB):
                lo = i * _P
                gi, p_i, rs_i = _attend(gi_f[lo:lo + _N],
                                        lhs_i[lo:lo + _N], d_i[i:i + 1, :_N])
                gh, p_h, rs_h = _attend(gh_f[lo:lo + _N],
                                        lhs_h[lo:lo + _N], d_h[i:i + 1, :_N])
                gs_scr[lo:lo + _N] = gi[:, :2 * _F] + gh[:, :2 * _F]
                gin_scr[lo:lo + _N] = gi[:, 2 * _F:3 * _F]
                ghn_scr[lo:lo + _N] = gh[:, 2 * _F:3 * _F]

                @pl.when(last)
                def _(i=i, l=l, p_i=p_i, rs_i=rs_i, p_h=p_h, rs_h=rs_h):
                    attn_i_ref[i, l] = p_i * rs_i
                    attn_h_ref[i, l] = p_h * rs_h
            s = jax.nn.sigmoid(gs_scr[...])
            r = s[:, :_F]
            z = s[:, _F:]
            n = jnp.tanh(gin_scr[...] + r * ghn_scr[...])
            h_new = (1.0 - z) * n + z * hfull
            h_scr[l] = h_new
            for i in range(_BB):
                lo = i * _P
                if l == _L - 1:
                    out_ref[i, t] = h_new[lo:lo + _N]

                @pl.when(last)
                def _(i=i, l=l, lo=lo, h_new=h_new):
                    hid_ref[i, l] = h_new[lo:lo + _N]
            x2 = h_new
        return carry

    jax.lax.fori_loop(0, _T, _step, 0)


def kernel(input, Wi, Wh, ai, ah):
    Bb, Tt, Nn, Ff = input.shape
    Ll = Wi.shape[0]
    nb = Bb // _BB

    wai = jnp.einsum('lfg,lag->lfa', Wi, ai)
    wah = jnp.einsum('lfg,lag->lfa', Wh, ah)
    wie = jnp.concatenate([Wi, wai], axis=2)
    whe = jnp.concatenate([Wh, wah], axis=2)

    grid = (nb,)
    out_shapes = (
        jax.ShapeDtypeStruct((Bb, Tt, Nn, Ff), jnp.float32),
        jax.ShapeDtypeStruct((Bb, Ll, Nn, Ff), jnp.float32),
        jax.ShapeDtypeStruct((Bb, Ll, Nn, Nn), jnp.float32),
        jax.ShapeDtypeStruct((Bb, Ll, Nn, Nn), jnp.float32),
    )
    in_specs = [
        pl.BlockSpec((_BB, Tt, Nn, Ff), lambda b: (b, 0, 0, 0)),
        pl.BlockSpec((Ll, Ff, 3 * Ff + 2), lambda b: (0, 0, 0)),
        pl.BlockSpec((Ll, Ff, 3 * Ff + 2), lambda b: (0, 0, 0)),
    ]
    out_specs = (
        pl.BlockSpec((_BB, Tt, Nn, Ff), lambda b: (b, 0, 0, 0)),
        pl.BlockSpec((_BB, Ll, Nn, Ff), lambda b: (b, 0, 0, 0)),
        pl.BlockSpec((_BB, Ll, Nn, Nn), lambda b: (b, 0, 0, 0)),
        pl.BlockSpec((_BB, Ll, Nn, Nn), lambda b: (b, 0, 0, 0)),
    )
    output, hidden, attn_i, attn_h = pl.pallas_call(
        _garnn_kernel,
        grid=grid,
        in_specs=in_specs,
        out_specs=out_specs,
        out_shape=out_shapes,
        scratch_shapes=[pltpu.VMEM((Ll, _M, Ff), jnp.float32),
                        pltpu.VMEM((_M, Ff), jnp.float32),
                        pltpu.VMEM((_M, 2 * Ff), jnp.float32),
                        pltpu.VMEM((_M, Ff), jnp.float32),
                        pltpu.VMEM((_M, Ff), jnp.float32)],
        compiler_params=pltpu.CompilerParams(
            dimension_semantics=("parallel",),
        ),
    )(input, wie, whe)
    return (output, hidden, attn_i, attn_h)


# speedup vs baseline: 1.0085x; 1.0054x over previous
import jax
import jax.numpy as jnp
from jax.experimental import pallas as pl
from jax.experimental.pallas import tpu as pltpu

_B, _T, _N, _F, _L = 32, 12, 207, 64, 2
_P = 208
_BB = 4
_M = _BB * _P


def _attend(h, lhs, d_row):
    rhs = jnp.concatenate(
        [jnp.ones((1, _N), dtype=jnp.float32), d_row], axis=0)
    e = jnp.dot(lhs, rhs, preferred_element_type=jnp.float32)
    e = jnp.maximum(e, 0.2 * e)
    p = jnp.exp2(e)
    rs = 1.0 / jnp.sum(p, axis=-1, keepdims=True)
    out = jnp.dot(p, h, preferred_element_type=jnp.float32) * rs
    return out, p, rs


def _garnn_kernel(x_ref, wi_ref, wh_ref,
                  out_ref, hid_ref, attn_i_ref, attn_h_ref,
                  h_scr, x2_scr, gs_scr, gin_scr, ghn_scr):
    h_scr[...] = jnp.zeros_like(h_scr)
    x2_scr[...] = jnp.zeros_like(x2_scr)

    def _step(t, carry):
        last = t == _T - 1
        for i in range(_BB):
            x2_scr[i * _P:i * _P + _N] = x_ref[i, t]
        x2 = x2_scr[...]
        for l in range(_L):
            hfull = h_scr[l]
            gi_f = jnp.dot(x2, wi_ref[l], preferred_element_type=jnp.float32)
            gh_f = jnp.dot(hfull, wh_ref[l], preferred_element_type=jnp.float32)
            e2i = gi_f[:, 3 * _F:3 * _F + 2]
            e2h = gh_f[:, 3 * _F:3 * _F + 2]
            ones_m = jnp.ones((_M, 1), dtype=jnp.float32)
            lhs_i = jnp.concatenate([e2i[:, 0:1], ones_m], axis=1)
            lhs_h = jnp.concatenate([e2h[:, 0:1], ones_m], axis=1)
            d_i = e2i[:, 1:2].reshape(_BB, _P)
            d_h = e2h[:, 1:2].reshape(_BB, _P)
            for i in range(_BB):
                lo = i * _P
                gi, p_i, rs_i = _attend(gi_f[lo:lo + _N],
                                        lhs_i[lo:lo + _N], d_i[i:i + 1, :_N])
                gh, p_h, rs_h = _attend(gh_f[lo:lo + _N],
                                        lhs_h[lo:lo + _N], d_h[i:i + 1, :_N])
                gs_scr[lo:lo + _N] = gi[:, :2 * _F] + gh[:, :2 * _F]
                gin_scr[lo:lo + _N] = gi[:, 2 * _F:3 * _F]
                ghn_scr[lo:lo + _N] = gh[:, 2 * _F:3 * _F]

                @pl.when(last)
                def _(i=i, l=l, p_i=p_i, rs_i=rs_i, p_h=p_h, rs_h=rs_h):
                    attn_i_ref[i, l] = p_i * rs_i
                    attn_h_ref[i, l] = p_h * rs_h
            s = jax.nn.sigmoid(gs_scr[...])
            r = s[:, :_F]
            z = s[:, _F:]
            n = jnp.tanh(gin_scr[...] + r * ghn_scr[...])
            h_new = (1.0 - z) * n + z * hfull
            h_scr[l] = h_new
            for i in range(_BB):
                lo = i * _P
                if l == _L - 1:
                    out_ref[i, t] = h_new[lo:lo + _N]

                @pl.when(last)
                def _(i=i, l=l, lo=lo, h_new=h_new):
                    hid_ref[i, l] = h_new[lo:lo + _N]
            x2 = h_new
        return carry

    jax.lax.fori_loop(0, _T, _step, 0)


def kernel(input, Wi, Wh, ai, ah):
    Bb, Tt, Nn, Ff = input.shape
    Ll = Wi.shape[0]
    nb = Bb // _BB

    log2e = jnp.float32(1.4426950408889634)
    wai = jnp.einsum('lfg,lag->lfa', Wi, ai) * log2e
    wah = jnp.einsum('lfg,lag->lfa', Wh, ah) * log2e
    wie = jnp.concatenate([Wi, wai], axis=2)
    whe = jnp.concatenate([Wh, wah], axis=2)

    grid = (nb,)
    out_shapes = (
        jax.ShapeDtypeStruct((Bb, Tt, Nn, Ff), jnp.float32),
        jax.ShapeDtypeStruct((Bb, Ll, Nn, Ff), jnp.float32),
        jax.ShapeDtypeStruct((Bb, Ll, Nn, Nn), jnp.float32),
        jax.ShapeDtypeStruct((Bb, Ll, Nn, Nn), jnp.float32),
    )
    in_specs = [
        pl.BlockSpec((_BB, Tt, Nn, Ff), lambda b: (b, 0, 0, 0)),
        pl.BlockSpec((Ll, Ff, 3 * Ff + 2), lambda b: (0, 0, 0)),
        pl.BlockSpec((Ll, Ff, 3 * Ff + 2), lambda b: (0, 0, 0)),
    ]
    out_specs = (
        pl.BlockSpec((_BB, Tt, Nn, Ff), lambda b: (b, 0, 0, 0)),
        pl.BlockSpec((_BB, Ll, Nn, Ff), lambda b: (b, 0, 0, 0)),
        pl.BlockSpec((_BB, Ll, Nn, Nn), lambda b: (b, 0, 0, 0)),
        pl.BlockSpec((_BB, Ll, Nn, Nn), lambda b: (b, 0, 0, 0)),
    )
    output, hidden, attn_i, attn_h = pl.pallas_call(
        _garnn_kernel,
        grid=grid,
        in_specs=in_specs,
        out_specs=out_specs,
        out_shape=out_shapes,
        scratch_shapes=[pltpu.VMEM((Ll, _M, Ff), jnp.float32),
                        pltpu.VMEM((_M, Ff), jnp.float32),
                        pltpu.VMEM((_M, 2 * Ff), jnp.float32),
                        pltpu.VMEM((_M, Ff), jnp.float32),
                        pltpu.VMEM((_M, Ff), jnp.float32)],
        compiler_params=pltpu.CompilerParams(
            dimension_semantics=("parallel",),
        ),
    )(input, wie, whe)
    return (output, hidden, attn_i, attn_h)
